# Initial kernel scaffold; baseline (speedup 1.0000x reference)
import jax
import jax.numpy as jnp
from jax.experimental import pallas as pl


def kernel(x, codebook, Wp, bp, Wr, br):
    raise NotImplementedError("write your pallas kernel here")


# speedup vs baseline: 3.5955x; 3.5955x over previous
import jax
import jax.numpy as jnp
from jax import lax
from jax.experimental import pallas as pl

_TILE = 256
_K = 512
_D = 64
_S = 32


def _fused_body(x_ref, cbt_ref, cb_ref, wp_ref, bp_ref, wr_ref, br_ref,
                rec_ref, idx_ref, codes_ref):
    x = x_ref[...]
    cbt = cbt_ref[...]
    cb = cb_ref[...]

    cn = jnp.sum(cbt * cbt, axis=0, keepdims=True)
    xc = lax.dot_general(x, cbt, (((1,), (0,)), ((), ())),
                         precision=lax.Precision.HIGHEST,
                         preferred_element_type=jnp.float32)
    d = cn - 2.0 * xc

    dmin = jnp.min(d, axis=1, keepdims=True)
    iota = lax.broadcasted_iota(jnp.int32, d.shape, 1)
    idx = jnp.min(jnp.where(d == dmin, iota, _K), axis=1)
    idx_ref[...] = idx

    onehot = (iota == idx[:, None]).astype(jnp.float32)
    z_q = lax.dot_general(onehot, cb, (((1,), (0,)), ((), ())),
                          precision=lax.Precision.HIGHEST,
                          preferred_element_type=jnp.float32)

    pq = x + (z_q - x)
    r = x - pq
    s = lax.dot_general(r, wp_ref[...], (((1,), (0,)), ((), ())),
                        preferred_element_type=jnp.float32) + bp_ref[...]
    codes = (s > 0).astype(jnp.float32)
    codes_ref[...] = codes
    q = 2.0 * codes - 1.0
    bsq = lax.dot_general(q, wr_ref[...], (((1,), (0,)), ((), ())),
                          preferred_element_type=jnp.float32) + br_ref[...]
    aq = r + (bsq - r)
    rec = pq + aq
    rec_ref[...] = x + (rec - x)


def kernel(x, codebook, Wp, bp, Wr, br):
    B, T, D = x.shape
    N = B * T
    x2 = x.reshape(N, D)
    cbt = codebook.T
    bp2 = bp.reshape(1, _S)
    br2 = br.reshape(1, _D)

    grid = (N // _TILE,)
    rec, idx, codes = pl.pallas_call(
        _fused_body,
        grid=grid,
        in_specs=[
            pl.BlockSpec((_TILE, D), lambda i: (i, 0)),
            pl.BlockSpec((D, _K), lambda i: (0, 0)),
            pl.BlockSpec((_K, D), lambda i: (0, 0)),
            pl.BlockSpec((D, _S), lambda i: (0, 0)),
            pl.BlockSpec((1, _S), lambda i: (0, 0)),
            pl.BlockSpec((_S, D), lambda i: (0, 0)),
            pl.BlockSpec((1, D), lambda i: (0, 0)),
        ],
        out_specs=[
            pl.BlockSpec((_TILE, D), lambda i: (i, 0)),
            pl.BlockSpec((_TILE,), lambda i: (i,)),
            pl.BlockSpec((_TILE, _S), lambda i: (i, 0)),
        ],
        out_shape=[
            jax.ShapeDtypeStruct((N, D), jnp.float32),
            jax.ShapeDtypeStruct((N,), jnp.int32),
            jax.ShapeDtypeStruct((N, _S), jnp.float32),
        ],
    )(x2, cbt, codebook, Wp, bp2, Wr, br2)

    return (rec.reshape(B, T, D), idx.reshape(B, T), codes.reshape(B, T, _S))


# speedup vs baseline: 4.6730x; 1.2997x over previous
import jax
import jax.numpy as jnp
from jax import lax
from jax.experimental import pallas as pl

_TILE = 512
_K = 512
_D = 64
_S = 32


def _fused_body(x_ref, cbt_ref, cbh_ref, cbm_ref, cbl_ref, wp_ref, bp_ref,
                wr_ref, br_ref, rec_ref, idx_ref, codes_ref):
    x = x_ref[...]
    cbt = cbt_ref[...]

    cn = jnp.sum(cbt * cbt, axis=0, keepdims=True)
    xc = lax.dot_general(x, cbt, (((1,), (0,)), ((), ())),
                         precision=lax.Precision.HIGHEST,
                         preferred_element_type=jnp.float32)
    d = cn - 2.0 * xc

    dmin = jnp.min(d, axis=1, keepdims=True)
    iota = lax.broadcasted_iota(jnp.int32, d.shape, 1)
    idx = jnp.min(jnp.where(d == dmin, iota, _K), axis=1)
    idx_ref[...] = idx

    onehot = (iota == idx[:, None]).astype(jnp.float32)
    dn = (((1,), (0,)), ((), ()))
    z_q = (lax.dot_general(onehot, cbh_ref[...], dn,
                           preferred_element_type=jnp.float32)
           + lax.dot_general(onehot, cbm_ref[...], dn,
                             preferred_element_type=jnp.float32)
           + lax.dot_general(onehot, cbl_ref[...], dn,
                             preferred_element_type=jnp.float32))

    pq = x + (z_q - x)
    r = x - pq
    s = lax.dot_general(r, wp_ref[...], (((1,), (0,)), ((), ())),
                        preferred_element_type=jnp.float32) + bp_ref[...]
    codes = (s > 0).astype(jnp.float32)
    codes_ref[...] = codes
    q = 2.0 * codes - 1.0
    bsq = lax.dot_general(q, wr_ref[...], (((1,), (0,)), ((), ())),
                          preferred_element_type=jnp.float32) + br_ref[...]
    aq = r + (bsq - r)
    rec = pq + aq
    rec_ref[...] = x + (rec - x)


def kernel(x, codebook, Wp, bp, Wr, br):
    B, T, D = x.shape
    N = B * T
    x2 = x.reshape(N, D)
    cbt = codebook.T
    cb_hi = codebook.astype(jnp.bfloat16).astype(jnp.float32)
    cb_mid = (codebook - cb_hi).astype(jnp.bfloat16).astype(jnp.float32)
    cb_lo = codebook - cb_hi - cb_mid
    bp2 = bp.reshape(1, _S)
    br2 = br.reshape(1, _D)

    grid = (N // _TILE,)
    rec, idx, codes = pl.pallas_call(
        _fused_body,
        grid=grid,
        in_specs=[
            pl.BlockSpec((_TILE, D), lambda i: (i, 0)),
            pl.BlockSpec((D, _K), lambda i: (0, 0)),
            pl.BlockSpec((_K, D), lambda i: (0, 0)),
            pl.BlockSpec((_K, D), lambda i: (0, 0)),
            pl.BlockSpec((_K, D), lambda i: (0, 0)),
            pl.BlockSpec((D, _S), lambda i: (0, 0)),
            pl.BlockSpec((1, _S), lambda i: (0, 0)),
            pl.BlockSpec((_S, D), lambda i: (0, 0)),
            pl.BlockSpec((1, D), lambda i: (0, 0)),
        ],
        out_specs=[
            pl.BlockSpec((_TILE, D), lambda i: (i, 0)),
            pl.BlockSpec((_TILE,), lambda i: (i,)),
            pl.BlockSpec((_TILE, _S), lambda i: (i, 0)),
        ],
        out_shape=[
            jax.ShapeDtypeStruct((N, D), jnp.float32),
            jax.ShapeDtypeStruct((N,), jnp.int32),
            jax.ShapeDtypeStruct((N, _S), jnp.float32),
        ],
    )(x2, cbt, cb_hi, cb_mid, cb_lo, Wp, bp2, Wr, br2)

    return (rec.reshape(B, T, D), idx.reshape(B, T), codes.reshape(B, T, _S))


# speedup vs baseline: 4.7498x; 1.0164x over previous
import jax
import jax.numpy as jnp
from jax import lax
from jax.experimental import pallas as pl

_TILE = 512
_K = 512
_D = 64
_S = 32


def _fused_body(x_ref, cb_ref, wp_ref, bp_ref, wr_ref, br_ref,
                rec_ref, idx_ref, codes_ref):
    x = x_ref[...]
    cb = cb_ref[...]

    cn = jnp.sum(cb * cb, axis=1, keepdims=True)
    cbaug = jnp.concatenate([cb, cn], axis=1)
    xaug = jnp.concatenate(
        [-2.0 * x, jnp.ones((x.shape[0], 1), jnp.float32)], axis=1)
    d = lax.dot_general(xaug, cbaug, (((1,), (1,)), ((), ())),
                        precision=lax.Precision.HIGHEST,
                        preferred_element_type=jnp.float32)

    dmin = jnp.min(d, axis=1, keepdims=True)
    iota = lax.broadcasted_iota(jnp.int32, d.shape, 1)
    idx = jnp.min(jnp.where(d == dmin, iota, _K), axis=1)
    idx_ref[...] = idx

    cb_hi = cb.astype(jnp.bfloat16).astype(jnp.float32)
    cb_mid = (cb - cb_hi).astype(jnp.bfloat16).astype(jnp.float32)
    cb_lo = cb - cb_hi - cb_mid
    onehot = (iota == idx[:, None]).astype(jnp.float32)
    dn = (((1,), (0,)), ((), ()))
    z_q = (lax.dot_general(onehot, cb_hi, dn,
                           preferred_element_type=jnp.float32)
           + lax.dot_general(onehot, cb_mid, dn,
                             preferred_element_type=jnp.float32)
           + lax.dot_general(onehot, cb_lo, dn,
                             preferred_element_type=jnp.float32))

    pq = x + (z_q - x)
    r = x - pq
    s = lax.dot_general(r, wp_ref[...], (((1,), (0,)), ((), ())),
                        preferred_element_type=jnp.float32) + bp_ref[...]
    codes = (s > 0).astype(jnp.float32)
    codes_ref[...] = codes
    q = 2.0 * codes - 1.0
    bsq = lax.dot_general(q, wr_ref[...], (((1,), (0,)), ((), ())),
                          preferred_element_type=jnp.float32) + br_ref[...]
    aq = r + (bsq - r)
    rec = pq + aq
    rec_ref[...] = x + (rec - x)


def kernel(x, codebook, Wp, bp, Wr, br):
    B, T, D = x.shape
    N = B * T
    x2 = x.reshape(N, D)
    bp2 = bp.reshape(1, _S)
    br2 = br.reshape(1, _D)

    grid = (N // _TILE,)
    rec, idx, codes = pl.pallas_call(
        _fused_body,
        grid=grid,
        in_specs=[
            pl.BlockSpec((_TILE, D), lambda i: (i, 0)),
            pl.BlockSpec((_K, D), lambda i: (0, 0)),
            pl.BlockSpec((D, _S), lambda i: (0, 0)),
            pl.BlockSpec((1, _S), lambda i: (0, 0)),
            pl.BlockSpec((_S, D), lambda i: (0, 0)),
            pl.BlockSpec((1, D), lambda i: (0, 0)),
        ],
        out_specs=[
            pl.BlockSpec((_TILE, D), lambda i: (i, 0)),
            pl.BlockSpec((_TILE,), lambda i: (i,)),
            pl.BlockSpec((_TILE, _S), lambda i: (i, 0)),
        ],
        out_shape=[
            jax.ShapeDtypeStruct((N, D), jnp.float32),
            jax.ShapeDtypeStruct((N,), jnp.int32),
            jax.ShapeDtypeStruct((N, _S), jnp.float32),
        ],
    )(x2, codebook, Wp, bp2, Wr, br2)

    return (rec.reshape(B, T, D), idx.reshape(B, T), codes.reshape(B, T, _S))


# speedup vs baseline: 5.0243x; 1.0578x over previous
import jax
import jax.numpy as jnp
from jax import lax
from jax.experimental import pallas as pl

_TILE = 1024
_K = 512
_D = 64
_S = 32


def _fused_body(x_ref, cb_ref, wp_ref, bp_ref, wr_ref, br_ref,
                rec_ref, idx_ref, codes_ref):
    x = x_ref[...]
    cb = cb_ref[...]

    cn = jnp.sum(cb * cb, axis=1, keepdims=True)
    cbaug = jnp.concatenate([cb, cn], axis=1)
    xaug = jnp.concatenate(
        [-2.0 * x, jnp.ones((x.shape[0], 1), jnp.float32)], axis=1)
    d = lax.dot_general(xaug, cbaug, (((1,), (1,)), ((), ())),
                        precision=lax.Precision.HIGHEST,
                        preferred_element_type=jnp.float32)

    dmin = jnp.min(d, axis=1, keepdims=True)
    iota = lax.broadcasted_iota(jnp.int32, d.shape, 1)
    idx = jnp.min(jnp.where(d == dmin, iota, _K), axis=1)
    idx_ref[...] = idx

    cb_hi = cb.astype(jnp.bfloat16).astype(jnp.float32)
    cb_mid = (cb - cb_hi).astype(jnp.bfloat16).astype(jnp.float32)
    cb_lo = cb - cb_hi - cb_mid
    onehot = (iota == idx[:, None]).astype(jnp.float32)
    dn = (((1,), (0,)), ((), ()))
    z_q = (lax.dot_general(onehot, cb_hi, dn,
                           preferred_element_type=jnp.float32)
           + lax.dot_general(onehot, cb_mid, dn,
                             preferred_element_type=jnp.float32)
           + lax.dot_general(onehot, cb_lo, dn,
                             preferred_element_type=jnp.float32))

    pq = x + (z_q - x)
    r = x - pq
    s = lax.dot_general(r, wp_ref[...], (((1,), (0,)), ((), ())),
                        preferred_element_type=jnp.float32) + bp_ref[...]
    codes = (s > 0).astype(jnp.float32)
    codes_ref[...] = codes
    q = 2.0 * codes - 1.0
    bsq = lax.dot_general(q, wr_ref[...], (((1,), (0,)), ((), ())),
                          preferred_element_type=jnp.float32) + br_ref[...]
    aq = r + (bsq - r)
    rec = pq + aq
    rec_ref[...] = x + (rec - x)


def kernel(x, codebook, Wp, bp, Wr, br):
    B, T, D = x.shape
    N = B * T
    x2 = x.reshape(N, D)
    bp2 = bp.reshape(1, _S)
    br2 = br.reshape(1, _D)

    grid = (N // _TILE,)
    rec, idx, codes = pl.pallas_call(
        _fused_body,
        grid=grid,
        in_specs=[
            pl.BlockSpec((_TILE, D), lambda i: (i, 0)),
            pl.BlockSpec((_K, D), lambda i: (0, 0)),
            pl.BlockSpec((D, _S), lambda i: (0, 0)),
            pl.BlockSpec((1, _S), lambda i: (0, 0)),
            pl.BlockSpec((_S, D), lambda i: (0, 0)),
            pl.BlockSpec((1, D), lambda i: (0, 0)),
        ],
        out_specs=[
            pl.BlockSpec((_TILE, D), lambda i: (i, 0)),
            pl.BlockSpec((_TILE,), lambda i: (i,)),
            pl.BlockSpec((_TILE, _S), lambda i: (i, 0)),
        ],
        out_shape=[
            jax.ShapeDtypeStruct((N, D), jnp.float32),
            jax.ShapeDtypeStruct((N,), jnp.int32),
            jax.ShapeDtypeStruct((N, _S), jnp.float32),
        ],
    )(x2, codebook, Wp, bp2, Wr, br2)

    return (rec.reshape(B, T, D), idx.reshape(B, T), codes.reshape(B, T, _S))


# speedup vs baseline: 5.1080x; 1.0167x over previous
import jax
import jax.numpy as jnp
from jax import lax
from jax.experimental import pallas as pl

_TILE = 2048
_K = 512
_D = 64
_S = 32


def _fused_body(x_ref, cb_ref, wp_ref, bp_ref, wr_ref, br_ref,
                rec_ref, idx_ref, codes_ref):
    x = x_ref[...]
    cb = cb_ref[...]

    cn = jnp.sum(cb * cb, axis=1, keepdims=True)
    cbaug = jnp.concatenate([cb, cn], axis=1)
    xaug = jnp.concatenate(
        [-2.0 * x, jnp.ones((x.shape[0], 1), jnp.float32)], axis=1)
    d = lax.dot_general(xaug, cbaug, (((1,), (1,)), ((), ())),
                        precision=lax.Precision.HIGHEST,
                        preferred_element_type=jnp.float32)

    dmin = jnp.min(d, axis=1, keepdims=True)
    iota = lax.broadcasted_iota(jnp.int32, d.shape, 1)
    idx = jnp.min(jnp.where(d == dmin, iota, _K), axis=1)
    idx_ref[...] = idx

    cb_hi = cb.astype(jnp.bfloat16).astype(jnp.float32)
    cb_mid = (cb - cb_hi).astype(jnp.bfloat16).astype(jnp.float32)
    cb_lo = cb - cb_hi - cb_mid
    onehot = (iota == idx[:, None]).astype(jnp.float32)
    dn = (((1,), (0,)), ((), ()))
    z_q = (lax.dot_general(onehot, cb_hi, dn,
                           preferred_element_type=jnp.float32)
           + lax.dot_general(onehot, cb_mid, dn,
                             preferred_element_type=jnp.float32)
           + lax.dot_general(onehot, cb_lo, dn,
                             preferred_element_type=jnp.float32))

    pq = x + (z_q - x)
    r = x - pq
    s = lax.dot_general(r, wp_ref[...], (((1,), (0,)), ((), ())),
                        preferred_element_type=jnp.float32) + bp_ref[...]
    codes = (s > 0).astype(jnp.float32)
    codes_ref[...] = codes
    q = 2.0 * codes - 1.0
    bsq = lax.dot_general(q, wr_ref[...], (((1,), (0,)), ((), ())),
                          preferred_element_type=jnp.float32) + br_ref[...]
    aq = r + (bsq - r)
    rec = pq + aq
    rec_ref[...] = x + (rec - x)


def kernel(x, codebook, Wp, bp, Wr, br):
    B, T, D = x.shape
    N = B * T
    x2 = x.reshape(N, D)
    bp2 = bp.reshape(1, _S)
    br2 = br.reshape(1, _D)

    grid = (N // _TILE,)
    rec, idx, codes = pl.pallas_call(
        _fused_body,
        grid=grid,
        in_specs=[
            pl.BlockSpec((_TILE, D), lambda i: (i, 0)),
            pl.BlockSpec((_K, D), lambda i: (0, 0)),
            pl.BlockSpec((D, _S), lambda i: (0, 0)),
            pl.BlockSpec((1, _S), lambda i: (0, 0)),
            pl.BlockSpec((_S, D), lambda i: (0, 0)),
            pl.BlockSpec((1, D), lambda i: (0, 0)),
        ],
        out_specs=[
            pl.BlockSpec((_TILE, D), lambda i: (i, 0)),
            pl.BlockSpec((_TILE,), lambda i: (i,)),
            pl.BlockSpec((_TILE, _S), lambda i: (i, 0)),
        ],
        out_shape=[
            jax.ShapeDtypeStruct((N, D), jnp.float32),
            jax.ShapeDtypeStruct((N,), jnp.int32),
            jax.ShapeDtypeStruct((N, _S), jnp.float32),
        ],
    )(x2, codebook, Wp, bp2, Wr, br2)

    return (rec.reshape(B, T, D), idx.reshape(B, T), codes.reshape(B, T, _S))
